# baseline (speedup 1.0000x reference)
import functools

import jax
import jax.numpy as jnp
from jax import lax
from jax.experimental import pallas as pl
from jax.experimental.pallas import tpu as pltpu
from jax.experimental.pallas import tpu_sc as plsc

NUM_CLIENTS = 1024
NUM_CLUSTERS = 64
PCA_DIM = 4
SINKHORN_REG = 0.2
TEMPERATURE = 0.5

NUM_SC_CORES = 1
NUM_SC_SUBCORES = 16
LANES = 16
ROWS_PER_SUBCORE = NUM_CLIENTS // NUM_SC_SUBCORES
CHUNKS = NUM_CLUSTERS // LANES


def _newton_sqrt(x, iters=16):
    y = 0.5 * (1.0 + x)
    for _ in range(iters):
        y = 0.5 * (y + x / y)
    return jnp.where(x > 0, y, 0.0)


def _sc_balance_body(ct_hbm, lg_hbm, out_hbm, ct_v, lg_v, cnt_v, sums_v,
                     out_v, shared):
    sid = lax.axis_index("s")
    cid = lax.axis_index("c")
    pltpu.sync_copy(ct_hbm, ct_v)
    pltpu.sync_copy(lg_hbm.at[pl.ds(sid * ROWS_PER_SUBCORE,
                                    ROWS_PER_SUBCORE)], lg_v)

    lane_iota = lax.iota(jnp.int32, LANES)
    iotas = [lane_iota + jnp.int32(c * LANES) for c in range(CHUNKS)]

    two_cn = []
    for c in range(CHUNKS):
        cn2 = None
        for j in range(PCA_DIM):
            row = ct_v[j, pl.ds(c * LANES, LANES)]
            sq = row * row
            cn2 = sq if cn2 is None else cn2 + sq
        two_cn.append(2.0 * _newton_sqrt(cn2))

    acc = [jnp.zeros((LANES,), jnp.float32) for _ in range(CHUNKS)]
    for c in range(CHUNKS):
        cnt_v[pl.ds(c * LANES, LANES)] = jnp.zeros((LANES,), jnp.float32)
    for r in range(ROWS_PER_SUBCORE):
        chunks = [lg_v[r, pl.ds(c * LANES, LANES)] - two_cn[c]
                  for c in range(CHUNKS)]
        vmax = chunks[0]
        for c in range(1, CHUNKS):
            vmax = jnp.maximum(vmax, chunks[c])
        rowmax = jnp.max(vmax)
        eqs = [jnp.where(chunks[c] == rowmax, 1.0, 0.0)
               for c in range(CHUNKS)]
        for c in range(CHUNKS):
            acc[c] = acc[c] + eqs[c]
        nties = jnp.sum(eqs[0] + eqs[1] + eqs[2] + eqs[3])

        @pl.when(nties > 1.5)
        def _(chunks=chunks, eqs=eqs, rowmax=rowmax):
            cand = None
            for c in range(CHUNKS):
                iv = jnp.where(chunks[c] == rowmax, iotas[c],
                               jnp.int32(NUM_CLUSTERS))
                cand = iv if cand is None else jnp.minimum(cand, iv)
            idx = jnp.min(cand)
            for c in range(CHUNKS):
                fix = jnp.where(iotas[c] == idx, 1.0, 0.0) - eqs[c]
                cnt_v[pl.ds(c * LANES, LANES)] = (
                    cnt_v[pl.ds(c * LANES, LANES)] + fix)

    for c in range(CHUNKS):
        cnt_v[pl.ds(c * LANES, LANES)] = cnt_v[pl.ds(c * LANES, LANES)] + acc[c]

    pltpu.sync_copy(cnt_v, shared.at[sid])
    plsc.subcore_barrier()

    @pl.when(sid == 0)
    def _():
        pltpu.sync_copy(shared, sums_v)
        always = lane_iota >= 0

        def _splat(scalar):
            return jnp.where(always, scalar, 0.0)

        counts = []
        for c in range(CHUNKS):
            tot = None
            for w in range(NUM_SC_SUBCORES):
                v = sums_v[w, pl.ds(c * LANES, LANES)]
                tot = v if tot is None else tot + v
            counts.append(tot)
        total = float(NUM_CLIENTS)
        mean = float(NUM_CLIENTS) / NUM_CLUSTERS
        gv = jnp.zeros((LANES,), jnp.float32)
        vv = jnp.zeros((LANES,), jnp.float32)
        for c in range(CHUNKS):
            p = counts[c] * (1.0 / total)
            gv = gv + p * (1.0 - p)
            d = counts[c] - mean
            vv = vv + d * d
        gini_v = _splat(jnp.sum(gv))
        std_v = _newton_sqrt(_splat(jnp.sum(vv)) * (1.0 / NUM_CLUSTERS))
        imb_v = std_v * (1.0 / (mean + 1e-10))
        out_v[...] = 0.5 * gini_v + 0.8 * imb_v
        pltpu.sync_copy(out_v, out_hbm.at[cid])


_sc_balance = functools.partial(
    pl.kernel,
    out_type=jax.ShapeDtypeStruct((NUM_SC_CORES, LANES), jnp.float32),
    mesh=plsc.VectorSubcoreMesh(
        core_axis_name="c", subcore_axis_name="s",
        num_cores=NUM_SC_CORES, num_subcores=NUM_SC_SUBCORES),
    compiler_params=pltpu.CompilerParams(needs_layout_passes=False),
    scratch_types=[
        pltpu.VMEM((PCA_DIM, NUM_CLUSTERS), jnp.float32),
        pltpu.VMEM((ROWS_PER_SUBCORE, NUM_CLUSTERS), jnp.float32),
        pltpu.VMEM((NUM_CLUSTERS,), jnp.float32),
        pltpu.VMEM((NUM_SC_SUBCORES, NUM_CLUSTERS), jnp.float32),
        pltpu.VMEM((LANES,), jnp.float32),
        pltpu.VMEM_SHARED((NUM_SC_SUBCORES, NUM_CLUSTERS), jnp.float32),
    ],
)(_sc_balance_body)


def _vwc_body(centers_ref, ct_ref, logits_ref, probs_ref, part_ref):
    c = centers_ref[...]
    ct = ct_ref[...]
    lg = logits_ref[...]

    cn2_row = jnp.sum(ct * ct, axis=0, keepdims=True)
    cn_row = jnp.where(cn2_row > 0,
                       jnp.sqrt(jnp.where(cn2_row > 0, cn2_row, 1.0)), 0.0)

    a = lg - (1.0 / TEMPERATURE) * cn_row
    m = jnp.max(a, axis=1, keepdims=True)
    e = jnp.exp(a - m)
    s = jnp.sum(e, axis=1, keepdims=True)
    probs = e / s
    probs_ref[...] = probs

    colsum = jnp.sum(probs, axis=0, keepdims=True)
    cluster_probs = colsum * (1.0 / NUM_CLIENTS)
    entropy = -jnp.sum(cluster_probs * jnp.log(cluster_probs + 1e-10))

    cn2_col = jnp.sum(c * c, axis=1, keepdims=True)
    gram = jnp.dot(c, ct, preferred_element_type=jnp.float32)
    pd2 = cn2_col + cn2_row - 2.0 * gram
    pd = jnp.where(pd2 > 0, jnp.sqrt(jnp.where(pd2 > 0, pd2, 1.0)), 0.0)
    iota_r = jax.lax.broadcasted_iota(jnp.int32, (NUM_CLUSTERS, NUM_CLUSTERS), 0)
    iota_c = jax.lax.broadcasted_iota(jnp.int32, (NUM_CLUSTERS, NUM_CLUSTERS), 1)
    pd = pd + jnp.where(iota_r == iota_c, 1e10, 0.0)
    min_dist = -jnp.min(pd)

    distance_loss = jnp.sum(colsum * cn_row)
    part_ref[0, 0] = (distance_loss - SINKHORN_REG * entropy
                      + 0.2 * min_dist)


def kernel(proxy_points, centers, logits):
    del proxy_points
    ct = centers.T
    balance = _sc_balance(ct, logits)
    probs, part = pl.pallas_call(
        _vwc_body,
        out_shape=(
            jax.ShapeDtypeStruct((NUM_CLIENTS, NUM_CLUSTERS), jnp.float32),
            jax.ShapeDtypeStruct((1, 1), jnp.float32),
        ),
        out_specs=(
            pl.BlockSpec(memory_space=pltpu.VMEM),
            pl.BlockSpec(memory_space=pltpu.SMEM),
        ),
        in_specs=(
            pl.BlockSpec(memory_space=pltpu.VMEM),
            pl.BlockSpec(memory_space=pltpu.VMEM),
            pl.BlockSpec(memory_space=pltpu.VMEM),
        ),
    )(centers, ct, logits)
    loss = part.reshape(()) + balance[0, 0]
    return probs, loss


# speedup vs baseline: 1.2561x; 1.2561x over previous
import functools

import jax
import jax.numpy as jnp
from jax import lax
from jax.experimental import pallas as pl
from jax.experimental.pallas import tpu as pltpu
from jax.experimental.pallas import tpu_sc as plsc

NUM_CLIENTS = 1024
NUM_CLUSTERS = 64
PCA_DIM = 4
SINKHORN_REG = 0.2
TEMPERATURE = 0.5

NUM_SC_CORES = 1
NUM_SC_SUBCORES = 16
LANES = 16
ROWS_PER_SUBCORE = NUM_CLIENTS // NUM_SC_SUBCORES
CHUNKS = NUM_CLUSTERS // LANES


def _newton_sqrt(x, iters=16):
    y = 0.5 * (1.0 + x)
    for _ in range(iters):
        y = 0.5 * (y + x / y)
    return jnp.where(x > 0, y, 0.0)


def _sc_balance_body(ct_hbm, lg_hbm, out_hbm, ct_v, lg_v, cnt_v, sums_v,
                     out_v, shared):
    sid = lax.axis_index("s")
    cid = lax.axis_index("c")
    pltpu.sync_copy(ct_hbm, ct_v)
    pltpu.sync_copy(lg_hbm.at[pl.ds(sid * ROWS_PER_SUBCORE,
                                    ROWS_PER_SUBCORE)], lg_v)

    lane_iota = lax.iota(jnp.int32, LANES)
    iotas = [lane_iota + jnp.int32(c * LANES) for c in range(CHUNKS)]

    two_cn = []
    for c in range(CHUNKS):
        cn2 = None
        for j in range(PCA_DIM):
            row = ct_v[j, pl.ds(c * LANES, LANES)]
            sq = row * row
            cn2 = sq if cn2 is None else cn2 + sq
        two_cn.append(2.0 * _newton_sqrt(cn2))

    acc = [jnp.zeros((LANES,), jnp.float32) for _ in range(CHUNKS)]
    for r in range(ROWS_PER_SUBCORE):
        chunks = [lg_v[r, pl.ds(c * LANES, LANES)] - two_cn[c]
                  for c in range(CHUNKS)]
        vmax = chunks[0]
        for c in range(1, CHUNKS):
            vmax = jnp.maximum(vmax, chunks[c])
        rowmax = jnp.max(vmax)
        cand = None
        for c in range(CHUNKS):
            iv = jnp.where(chunks[c] == rowmax, iotas[c],
                           jnp.int32(NUM_CLUSTERS))
            cand = iv if cand is None else jnp.minimum(cand, iv)
        idx = jnp.min(cand)
        for c in range(CHUNKS):
            acc[c] = acc[c] + (iotas[c] == idx).astype(jnp.float32)

    for c in range(CHUNKS):
        cnt_v[pl.ds(c * LANES, LANES)] = acc[c]

    pltpu.sync_copy(cnt_v, shared.at[sid])
    plsc.subcore_barrier()

    @pl.when(sid == 0)
    def _():
        pltpu.sync_copy(shared, sums_v)
        always = lane_iota >= 0

        def _splat(scalar):
            return jnp.where(always, scalar, 0.0)

        counts = []
        for c in range(CHUNKS):
            tot = None
            for w in range(NUM_SC_SUBCORES):
                v = sums_v[w, pl.ds(c * LANES, LANES)]
                tot = v if tot is None else tot + v
            counts.append(tot)
        total = float(NUM_CLIENTS)
        mean = float(NUM_CLIENTS) / NUM_CLUSTERS
        gv = jnp.zeros((LANES,), jnp.float32)
        vv = jnp.zeros((LANES,), jnp.float32)
        for c in range(CHUNKS):
            p = counts[c] * (1.0 / total)
            gv = gv + p * (1.0 - p)
            d = counts[c] - mean
            vv = vv + d * d
        gini_v = _splat(jnp.sum(gv))
        std_v = _newton_sqrt(_splat(jnp.sum(vv)) * (1.0 / NUM_CLUSTERS))
        imb_v = std_v * (1.0 / (mean + 1e-10))
        out_v[...] = 0.5 * gini_v + 0.8 * imb_v
        pltpu.sync_copy(out_v, out_hbm.at[cid])


_sc_balance = functools.partial(
    pl.kernel,
    out_type=jax.ShapeDtypeStruct((NUM_SC_CORES, LANES), jnp.float32),
    mesh=plsc.VectorSubcoreMesh(
        core_axis_name="c", subcore_axis_name="s",
        num_cores=NUM_SC_CORES, num_subcores=NUM_SC_SUBCORES),
    compiler_params=pltpu.CompilerParams(needs_layout_passes=False),
    scratch_types=[
        pltpu.VMEM((PCA_DIM, NUM_CLUSTERS), jnp.float32),
        pltpu.VMEM((ROWS_PER_SUBCORE, NUM_CLUSTERS), jnp.float32),
        pltpu.VMEM((NUM_CLUSTERS,), jnp.float32),
        pltpu.VMEM((NUM_SC_SUBCORES, NUM_CLUSTERS), jnp.float32),
        pltpu.VMEM((LANES,), jnp.float32),
        pltpu.VMEM_SHARED((NUM_SC_SUBCORES, NUM_CLUSTERS), jnp.float32),
    ],
)(_sc_balance_body)


def _vwc_body(centers_ref, ct_ref, logits_ref, probs_ref, part_ref):
    c = centers_ref[...]
    ct = ct_ref[...]
    lg = logits_ref[...]

    cn2_row = jnp.sum(ct * ct, axis=0, keepdims=True)
    cn_row = jnp.where(cn2_row > 0,
                       jnp.sqrt(jnp.where(cn2_row > 0, cn2_row, 1.0)), 0.0)

    a = lg - (1.0 / TEMPERATURE) * cn_row
    m = jnp.max(a, axis=1, keepdims=True)
    e = jnp.exp(a - m)
    s = jnp.sum(e, axis=1, keepdims=True)
    probs = e / s
    probs_ref[...] = probs

    colsum = jnp.sum(probs, axis=0, keepdims=True)
    cluster_probs = colsum * (1.0 / NUM_CLIENTS)
    entropy = -jnp.sum(cluster_probs * jnp.log(cluster_probs + 1e-10))

    cn2_col = jnp.sum(c * c, axis=1, keepdims=True)
    gram = jnp.dot(c, ct, preferred_element_type=jnp.float32)
    pd2 = cn2_col + cn2_row - 2.0 * gram
    pd = jnp.where(pd2 > 0, jnp.sqrt(jnp.where(pd2 > 0, pd2, 1.0)), 0.0)
    iota_r = jax.lax.broadcasted_iota(jnp.int32, (NUM_CLUSTERS, NUM_CLUSTERS), 0)
    iota_c = jax.lax.broadcasted_iota(jnp.int32, (NUM_CLUSTERS, NUM_CLUSTERS), 1)
    pd = pd + jnp.where(iota_r == iota_c, 1e10, 0.0)
    min_dist = -jnp.min(pd)

    distance_loss = jnp.sum(colsum * cn_row)
    part_ref[0, 0] = (distance_loss - SINKHORN_REG * entropy
                      + 0.2 * min_dist)


def kernel(proxy_points, centers, logits):
    del proxy_points
    ct = centers.T
    balance = _sc_balance(ct, logits)
    probs, part = pl.pallas_call(
        _vwc_body,
        out_shape=(
            jax.ShapeDtypeStruct((NUM_CLIENTS, NUM_CLUSTERS), jnp.float32),
            jax.ShapeDtypeStruct((1, 1), jnp.float32),
        ),
        out_specs=(
            pl.BlockSpec(memory_space=pltpu.VMEM),
            pl.BlockSpec(memory_space=pltpu.SMEM),
        ),
        in_specs=(
            pl.BlockSpec(memory_space=pltpu.VMEM),
            pl.BlockSpec(memory_space=pltpu.VMEM),
            pl.BlockSpec(memory_space=pltpu.VMEM),
        ),
    )(centers, ct, logits)
    loss = part.reshape(()) + balance[0, 0]
    return probs, loss
